# baseline (speedup 1.0000x reference)
import functools

import jax
import jax.numpy as jnp
from jax import lax
from jax.experimental import pallas as pl
from jax.experimental.pallas import tpu as pltpu
from jax.experimental.pallas import tpu_sc as plsc

N = 10000
E = 160000
H = 512
C = 128
Np = 10240
TILES = 16
EP = E // TILES
B = 80
NB = EP // B
STRIPE = Np // TILES
OB = 40
R = 2000
EPS = 1e-5


def _make_segsum(nc):
    npc = nc // 2

    def body(xcf, pk3, zrow, out, pk, s80a, d80a, s80b, d80b,
             rows0, rows1, acc, sem0, sem1, semS0, semS1):
        c = lax.axis_index("c")
        s = lax.axis_index("s")

        pltpu.sync_copy(pk3.at[s], pk)

        def unpack(j, sref, dref):
            def u(l, _):
                v = pk[j, pl.ds(l * 16, 16)]
                sref[pl.ds(l * 16, 16)] = lax.bitwise_and(v, 0xFFFF)
                dref[pl.ds(l * 16, 16)] = lax.shift_right_logical(v, 16)
                return 0
            lax.fori_loop(0, B // 16, u, 0)

        for k in range(npc):
            q = c * npc + k

            pltpu.sync_copy(zrow, acc.at[pl.ds(s * STRIPE, STRIPE)])
            plsc.subcore_barrier()

            unpack(0, s80a, d80a)
            pltpu.async_copy(xcf.at[q].at[s80a], rows0, sem0)

            def pair(jj, _):
                j0 = 2 * jj
                unpack(j0 + 1, s80b, d80b)
                pltpu.async_copy(xcf.at[q].at[s80b], rows1, sem1)
                pltpu.make_async_copy(xcf.at[q].at[s80a], rows0, sem0).wait()
                sc0 = pltpu.async_copy(rows0, acc.at[d80a], semS0, add=True)
                pltpu.make_async_copy(xcf.at[q].at[s80b], rows1, sem1).wait()
                sc1 = pltpu.async_copy(rows1, acc.at[d80b], semS1, add=True)
                sc0.wait()

                @pl.when(j0 + 2 < NB)
                def _():
                    unpack(j0 + 2, s80a, d80a)
                    pltpu.async_copy(xcf.at[q].at[s80a], rows0, sem0)

                sc1.wait()
                return 0
            lax.fori_loop(0, NB // 2, pair, 0)

            pltpu.make_async_copy(xcf.at[q].at[s80a], rows0, sem0).wait()
            pltpu.sync_copy(rows0, acc.at[d80a], add=True)
            plsc.subcore_barrier()

            pltpu.sync_copy(acc.at[pl.ds(s * STRIPE, STRIPE)],
                            out.at[q, pl.ds(s * STRIPE, STRIPE)])

    return pl.kernel(
        body,
        out_type=jax.ShapeDtypeStruct((nc, Np, C), jnp.float32),
        mesh=plsc.VectorSubcoreMesh(
            core_axis_name="c", subcore_axis_name="s",
            num_cores=2, num_subcores=16),
        scratch_types=[
            pltpu.VMEM((NB, B), jnp.int32),
            pltpu.VMEM((B,), jnp.int32),
            pltpu.VMEM((B,), jnp.int32),
            pltpu.VMEM((B,), jnp.int32),
            pltpu.VMEM((B,), jnp.int32),
            pltpu.VMEM((B, C), jnp.float32),
            pltpu.VMEM((B, C), jnp.float32),
            pltpu.VMEM_SHARED((Np, C), jnp.float32),
            pltpu.SemaphoreType.DMA,
            pltpu.SemaphoreType.DMA,
            pltpu.SemaphoreType.DMA,
            pltpu.SemaphoreType.DMA,
        ],
    )


@functools.lru_cache(maxsize=None)
def _get_segsum(nc):
    return _make_segsum(nc)


def _mm_body(nc, x_ref, w_ref, o_ref):
    acc = jnp.dot(x_ref[0], w_ref[0], preferred_element_type=jnp.float32)
    for qq in range(1, nc):
        acc += jnp.dot(x_ref[qq], w_ref[qq], preferred_element_type=jnp.float32)
    o_ref[...] = acc


def _make_mm(nc):
    return pl.pallas_call(
        functools.partial(_mm_body, nc),
        grid=(N // R,),
        in_specs=[
            pl.BlockSpec((nc, R, C), lambda i: (0, i, 0)),
            pl.BlockSpec((nc, C, H), lambda i: (0, 0, 0)),
        ],
        out_specs=pl.BlockSpec((R, H), lambda i: (i, 0)),
        out_shape=jax.ShapeDtypeStruct((N, H), jnp.float32),
    )


_mm2 = _make_mm(2)
_mm4 = _make_mm(4)


def _py_body(nc, a_ref, x_ref, wl_ref, wr_ref, bl_ref, y_ref, st_ref):
    i = pl.program_id(0)
    acc = jnp.broadcast_to(bl_ref[...], (R, H))
    accr = jnp.zeros((R, H), jnp.float32)
    for qq in range(nc):
        acc = acc + jnp.dot(a_ref[qq], wl_ref[qq],
                            preferred_element_type=jnp.float32)
        accr = accr + jnp.dot(x_ref[qq], wr_ref[qq],
                              preferred_element_type=jnp.float32)
    y = jnp.maximum(acc * accr, 0.0)
    y_ref[...] = y
    s1 = jnp.sum(y, axis=0, keepdims=True)
    s2 = jnp.sum(y * y, axis=0, keepdims=True)
    mx = jnp.max(y, axis=0, keepdims=True)
    mn = jnp.min(y, axis=0, keepdims=True)

    @pl.when(i == 0)
    def _():
        st_ref[...] = jnp.concatenate([s1, s2, mx, mn], axis=0)

    @pl.when(i > 0)
    def _():
        prev = st_ref[...]
        st_ref[...] = jnp.concatenate(
            [prev[0:1] + s1, prev[1:2] + s2,
             jnp.maximum(prev[2:3], mx), jnp.minimum(prev[3:4], mn)], axis=0)


def _make_py(nc):
    return pl.pallas_call(
        functools.partial(_py_body, nc),
        grid=(N // R,),
        in_specs=[
            pl.BlockSpec((nc, R, C), lambda i: (0, i, 0)),
            pl.BlockSpec((nc, R, C), lambda i: (0, i, 0)),
            pl.BlockSpec((nc, C, H), lambda i: (0, 0, 0)),
            pl.BlockSpec((nc, C, H), lambda i: (0, 0, 0)),
            pl.BlockSpec((1, H), lambda i: (0, 0)),
        ],
        out_specs=[
            pl.BlockSpec((R, H), lambda i: (i, 0)),
            pl.BlockSpec((4, H), lambda i: (0, 0)),
        ],
        out_shape=[
            jax.ShapeDtypeStruct((N, H), jnp.float32),
            jax.ShapeDtypeStruct((4, H), jnp.float32),
        ],
    )


_py2 = _make_py(2)
_py4 = _make_py(4)


def _bn_body(y_ref, st_ref, g_ref, b_ref, o_ref):
    m = st_ref[0:1] / N
    var = st_ref[1:2] / N - m * m
    inv = g_ref[...] * lax.rsqrt(var + EPS)
    z = (y_ref[...] - m) * inv + b_ref[...]
    for qq in range(4):
        o_ref[qq] = z[:, qq * C:(qq + 1) * C]


_bn = pl.pallas_call(
    _bn_body,
    grid=(N // R,),
    in_specs=[
        pl.BlockSpec((R, H), lambda i: (i, 0)),
        pl.BlockSpec((4, H), lambda i: (0, 0)),
        pl.BlockSpec((1, H), lambda i: (0, 0)),
        pl.BlockSpec((1, H), lambda i: (0, 0)),
    ],
    out_specs=pl.BlockSpec((4, R, C), lambda i: (0, i, 0)),
    out_shape=jax.ShapeDtypeStruct((4, Np, C), jnp.float32),
)


def _expsum_body(y_ref, st_ref, g_ref, b_ref, e_ref, s_ref):
    i = pl.program_id(0)
    m = st_ref[0:1] / N
    var = st_ref[1:2] / N - m * m
    inv = g_ref[...] * lax.rsqrt(var + EPS)
    b = b_ref[...]
    zmax = jnp.where(inv >= 0.0,
                     (st_ref[2:3] - m) * inv, (st_ref[3:4] - m) * inv) + b
    z = (y_ref[...] - m) * inv + b
    e = jnp.exp(z - zmax)
    e_ref[...] = e
    s1 = jnp.sum(e, axis=0, keepdims=True)

    @pl.when(i == 0)
    def _():
        s_ref[...] = s1

    @pl.when(i > 0)
    def _():
        s_ref[...] = s_ref[...] + s1


_expsum = pl.pallas_call(
    _expsum_body,
    grid=(N // R,),
    in_specs=[
        pl.BlockSpec((R, H), lambda i: (i, 0)),
        pl.BlockSpec((4, H), lambda i: (0, 0)),
        pl.BlockSpec((1, H), lambda i: (0, 0)),
        pl.BlockSpec((1, H), lambda i: (0, 0)),
    ],
    out_specs=[
        pl.BlockSpec((R, H), lambda i: (i, 0)),
        pl.BlockSpec((1, H), lambda i: (0, 0)),
    ],
    out_shape=[
        jax.ShapeDtypeStruct((N, H), jnp.float32),
        jax.ShapeDtypeStruct((1, H), jnp.float32),
    ],
)


def _div_body(e_ref, s_ref, o_ref):
    o_ref[...] = e_ref[...] / s_ref[...]


_div = pl.pallas_call(
    _div_body,
    grid=(N // R,),
    in_specs=[
        pl.BlockSpec((R, H), lambda i: (i, 0)),
        pl.BlockSpec((1, H), lambda i: (0, 0)),
    ],
    out_specs=pl.BlockSpec((R, H), lambda i: (i, 0)),
    out_shape=jax.ShapeDtypeStruct((N, H), jnp.float32),
)


def kernel(node_feature, edge_index, global_x, Wl1, bl1, Wr1, g1, b1,
           Wl2, bl2, Wr2, g2, b2, Wl3, bl3, Wr3, g3, b3):
    del global_x
    src = edge_index[0].astype(jnp.int32)
    dst = edge_index[1].astype(jnp.int32)
    pk3 = ((dst << 16) | src).reshape(TILES, NB, B)
    zrow = jnp.zeros((STRIPE, C), jnp.float32)

    nf = jnp.pad(node_feature, ((0, Np - N), (0, 0)))
    xc1 = jnp.stack([nf[:, 0:C], nf[:, C:2 * C]], axis=0)

    aggr = _get_segsum(2)(xc1, pk3, zrow)
    y, st = _py2(aggr, xc1, Wl1.reshape(2, C, H), Wr1.reshape(2, C, H),
                 bl1.reshape(1, H))
    xc = _bn(y, st, g1.reshape(1, H), b1.reshape(1, H))

    aggr = _get_segsum(4)(xc, pk3, zrow)
    y, st = _py4(aggr, xc, Wl2.reshape(4, C, H), Wr2.reshape(4, C, H),
                 bl2.reshape(1, H))
    xc = _bn(y, st, g2.reshape(1, H), b2.reshape(1, H))

    aggr = _get_segsum(4)(xc, pk3, zrow)
    y, st = _py4(aggr, xc, Wl3.reshape(4, C, H), Wr3.reshape(4, C, H),
                 bl3.reshape(1, H))

    e, ssum = _expsum(y, st, g3.reshape(1, H), b3.reshape(1, H))
    return _div(e, ssum)


# speedup vs baseline: 1.2372x; 1.2372x over previous
import functools

import jax
import jax.numpy as jnp
from jax import lax
from jax.experimental import pallas as pl
from jax.experimental.pallas import tpu as pltpu
from jax.experimental.pallas import tpu_sc as plsc

N = 10000
E = 160000
H = 512
C = 128
Np = 10240
TILES = 16
EP = E // TILES
B = 80
NB = EP // B
STRIPE = Np // TILES
OB = 40
R = 2000
EPS = 1e-5


def _make_segsum(nc):
    npc = nc // 2

    def body(xcf, pk3, zrow, out, pk, s80a, d80a, s80b, d80b,
             rows0, rows1, acc, sem0, sem1):
        c = lax.axis_index("c")
        s = lax.axis_index("s")

        pltpu.sync_copy(pk3.at[s], pk)

        def unpack(j, sref, dref):
            def u(l, _):
                v = pk[j, pl.ds(l * 16, 16)]
                sref[pl.ds(l * 16, 16)] = lax.bitwise_and(v, 0xFFFF)
                dref[pl.ds(l * 16, 16)] = lax.shift_right_logical(v, 16)
                return 0
            lax.fori_loop(0, B // 16, u, 0)

        for k in range(npc):
            q = c * npc + k

            pltpu.sync_copy(zrow, acc.at[pl.ds(s * STRIPE, STRIPE)])
            plsc.subcore_barrier()

            unpack(0, s80a, d80a)
            pltpu.async_copy(xcf.at[q].at[s80a], rows0, sem0)

            def pair(jj, _):
                j0 = 2 * jj
                unpack(j0 + 1, s80b, d80b)
                pltpu.async_copy(xcf.at[q].at[s80b], rows1, sem1)
                pltpu.make_async_copy(xcf.at[q].at[s80a], rows0, sem0).wait()
                pltpu.sync_copy(rows0, acc.at[d80a], add=True)

                @pl.when(j0 + 2 < NB)
                def _():
                    unpack(j0 + 2, s80a, d80a)
                    pltpu.async_copy(xcf.at[q].at[s80a], rows0, sem0)

                pltpu.make_async_copy(xcf.at[q].at[s80b], rows1, sem1).wait()
                pltpu.sync_copy(rows1, acc.at[d80b], add=True)
                return 0
            lax.fori_loop(0, NB // 2, pair, 0)

            pltpu.make_async_copy(xcf.at[q].at[s80a], rows0, sem0).wait()
            pltpu.sync_copy(rows0, acc.at[d80a], add=True)
            plsc.subcore_barrier()

            pltpu.sync_copy(acc.at[pl.ds(s * STRIPE, STRIPE)],
                            out.at[q, pl.ds(s * STRIPE, STRIPE)])

    return pl.kernel(
        body,
        out_type=jax.ShapeDtypeStruct((nc, Np, C), jnp.float32),
        mesh=plsc.VectorSubcoreMesh(
            core_axis_name="c", subcore_axis_name="s",
            num_cores=2, num_subcores=16),
        scratch_types=[
            pltpu.VMEM((NB, B), jnp.int32),
            pltpu.VMEM((B,), jnp.int32),
            pltpu.VMEM((B,), jnp.int32),
            pltpu.VMEM((B,), jnp.int32),
            pltpu.VMEM((B,), jnp.int32),
            pltpu.VMEM((B, C), jnp.float32),
            pltpu.VMEM((B, C), jnp.float32),
            pltpu.VMEM_SHARED((Np, C), jnp.float32),
            pltpu.SemaphoreType.DMA,
            pltpu.SemaphoreType.DMA,
        ],
    )


@functools.lru_cache(maxsize=None)
def _get_segsum(nc):
    return _make_segsum(nc)


def _mm_body(nc, x_ref, w_ref, o_ref):
    acc = jnp.dot(x_ref[0], w_ref[0], preferred_element_type=jnp.float32)
    for qq in range(1, nc):
        acc += jnp.dot(x_ref[qq], w_ref[qq], preferred_element_type=jnp.float32)
    o_ref[...] = acc


def _make_mm(nc):
    return pl.pallas_call(
        functools.partial(_mm_body, nc),
        grid=(N // R,),
        in_specs=[
            pl.BlockSpec((nc, R, C), lambda i: (0, i, 0)),
            pl.BlockSpec((nc, C, H), lambda i: (0, 0, 0)),
        ],
        out_specs=pl.BlockSpec((R, H), lambda i: (i, 0)),
        out_shape=jax.ShapeDtypeStruct((N, H), jnp.float32),
    )


_mm2 = _make_mm(2)
_mm4 = _make_mm(4)


def _py_body(nc, a_ref, x_ref, wl_ref, wr_ref, bl_ref, y_ref, st_ref):
    i = pl.program_id(0)
    acc = jnp.broadcast_to(bl_ref[...], (R, H))
    accr = jnp.zeros((R, H), jnp.float32)
    for qq in range(nc):
        acc = acc + jnp.dot(a_ref[qq], wl_ref[qq],
                            preferred_element_type=jnp.float32)
        accr = accr + jnp.dot(x_ref[qq], wr_ref[qq],
                              preferred_element_type=jnp.float32)
    y = jnp.maximum(acc * accr, 0.0)
    y_ref[...] = y
    s1 = jnp.sum(y, axis=0, keepdims=True)
    s2 = jnp.sum(y * y, axis=0, keepdims=True)
    mx = jnp.max(y, axis=0, keepdims=True)
    mn = jnp.min(y, axis=0, keepdims=True)

    @pl.when(i == 0)
    def _():
        st_ref[...] = jnp.concatenate([s1, s2, mx, mn], axis=0)

    @pl.when(i > 0)
    def _():
        prev = st_ref[...]
        st_ref[...] = jnp.concatenate(
            [prev[0:1] + s1, prev[1:2] + s2,
             jnp.maximum(prev[2:3], mx), jnp.minimum(prev[3:4], mn)], axis=0)


def _make_py(nc):
    return pl.pallas_call(
        functools.partial(_py_body, nc),
        grid=(N // R,),
        in_specs=[
            pl.BlockSpec((nc, R, C), lambda i: (0, i, 0)),
            pl.BlockSpec((nc, R, C), lambda i: (0, i, 0)),
            pl.BlockSpec((nc, C, H), lambda i: (0, 0, 0)),
            pl.BlockSpec((nc, C, H), lambda i: (0, 0, 0)),
            pl.BlockSpec((1, H), lambda i: (0, 0)),
        ],
        out_specs=[
            pl.BlockSpec((R, H), lambda i: (i, 0)),
            pl.BlockSpec((4, H), lambda i: (0, 0)),
        ],
        out_shape=[
            jax.ShapeDtypeStruct((N, H), jnp.float32),
            jax.ShapeDtypeStruct((4, H), jnp.float32),
        ],
    )


_py2 = _make_py(2)
_py4 = _make_py(4)


def _bn_body(y_ref, st_ref, g_ref, b_ref, o_ref):
    m = st_ref[0:1] / N
    var = st_ref[1:2] / N - m * m
    inv = g_ref[...] * lax.rsqrt(var + EPS)
    z = (y_ref[...] - m) * inv + b_ref[...]
    for qq in range(4):
        o_ref[qq] = z[:, qq * C:(qq + 1) * C]


_bn = pl.pallas_call(
    _bn_body,
    grid=(N // R,),
    in_specs=[
        pl.BlockSpec((R, H), lambda i: (i, 0)),
        pl.BlockSpec((4, H), lambda i: (0, 0)),
        pl.BlockSpec((1, H), lambda i: (0, 0)),
        pl.BlockSpec((1, H), lambda i: (0, 0)),
    ],
    out_specs=pl.BlockSpec((4, R, C), lambda i: (0, i, 0)),
    out_shape=jax.ShapeDtypeStruct((4, Np, C), jnp.float32),
)


def _expsum_body(y_ref, st_ref, g_ref, b_ref, e_ref, s_ref):
    i = pl.program_id(0)
    m = st_ref[0:1] / N
    var = st_ref[1:2] / N - m * m
    inv = g_ref[...] * lax.rsqrt(var + EPS)
    b = b_ref[...]
    zmax = jnp.where(inv >= 0.0,
                     (st_ref[2:3] - m) * inv, (st_ref[3:4] - m) * inv) + b
    z = (y_ref[...] - m) * inv + b
    e = jnp.exp(z - zmax)
    e_ref[...] = e
    s1 = jnp.sum(e, axis=0, keepdims=True)

    @pl.when(i == 0)
    def _():
        s_ref[...] = s1

    @pl.when(i > 0)
    def _():
        s_ref[...] = s_ref[...] + s1


_expsum = pl.pallas_call(
    _expsum_body,
    grid=(N // R,),
    in_specs=[
        pl.BlockSpec((R, H), lambda i: (i, 0)),
        pl.BlockSpec((4, H), lambda i: (0, 0)),
        pl.BlockSpec((1, H), lambda i: (0, 0)),
        pl.BlockSpec((1, H), lambda i: (0, 0)),
    ],
    out_specs=[
        pl.BlockSpec((R, H), lambda i: (i, 0)),
        pl.BlockSpec((1, H), lambda i: (0, 0)),
    ],
    out_shape=[
        jax.ShapeDtypeStruct((N, H), jnp.float32),
        jax.ShapeDtypeStruct((1, H), jnp.float32),
    ],
)


def _div_body(e_ref, s_ref, o_ref):
    o_ref[...] = e_ref[...] / s_ref[...]


_div = pl.pallas_call(
    _div_body,
    grid=(N // R,),
    in_specs=[
        pl.BlockSpec((R, H), lambda i: (i, 0)),
        pl.BlockSpec((1, H), lambda i: (0, 0)),
    ],
    out_specs=pl.BlockSpec((R, H), lambda i: (i, 0)),
    out_shape=jax.ShapeDtypeStruct((N, H), jnp.float32),
)


def kernel(node_feature, edge_index, global_x, Wl1, bl1, Wr1, g1, b1,
           Wl2, bl2, Wr2, g2, b2, Wl3, bl3, Wr3, g3, b3):
    del global_x
    src = edge_index[0].astype(jnp.int32)
    dst = edge_index[1].astype(jnp.int32)
    pk3 = ((dst << 16) | src).reshape(TILES, NB, B)
    zrow = jnp.zeros((STRIPE, C), jnp.float32)

    nf = jnp.pad(node_feature, ((0, Np - N), (0, 0)))
    xc1 = jnp.stack([nf[:, 0:C], nf[:, C:2 * C]], axis=0)

    aggr = _get_segsum(2)(xc1, pk3, zrow)
    y, st = _py2(aggr, xc1, Wl1.reshape(2, C, H), Wr1.reshape(2, C, H),
                 bl1.reshape(1, H))
    xc = _bn(y, st, g1.reshape(1, H), b1.reshape(1, H))

    aggr = _get_segsum(4)(xc, pk3, zrow)
    y, st = _py4(aggr, xc, Wl2.reshape(4, C, H), Wr2.reshape(4, C, H),
                 bl2.reshape(1, H))
    xc = _bn(y, st, g2.reshape(1, H), b2.reshape(1, H))

    aggr = _get_segsum(4)(xc, pk3, zrow)
    y, st = _py4(aggr, xc, Wl3.reshape(4, C, H), Wr3.reshape(4, C, H),
                 bl3.reshape(1, H))

    e, ssum = _expsum(y, st, g3.reshape(1, H), b3.reshape(1, H))
    return _div(e, ssum)
